# baseline (speedup 1.0000x reference)
import jax
import jax.numpy as jnp
from jax import lax
from jax.experimental import pallas as pl
from jax.experimental.pallas import tpu as pltpu

N_TILE = 2048
N_FULL = 48
N_TAIL = 1696
MBUF = 3
NBUF = 4


def _body(x_ref, mem_ref, out_ref, m_v, mt_v,
          o0, o1, o2, o3, o_tail,
          in_sem, s0, s1, s2, s3, tail_sem):
    x = x_ref[...]
    obufs = (o0, o1, o2, o3)
    osems = (s0, s1, s2, s3)

    def in_copy(i, slot):
        return pltpu.make_async_copy(
            mem_ref.at[pl.ds(i * N_TILE, N_TILE), :], m_v.at[slot],
            in_sem.at[slot])

    def out_copy(i, slot):
        return pltpu.make_async_copy(
            obufs[slot], out_ref.at[:, pl.ds(i * N_TILE, N_TILE)],
            osems[slot])

    for s in range(MBUF):
        in_copy(s, s).start()

    def matmul(mslot):
        mt_v[...] = m_v[mslot].T
        return lax.dot_general(
            x, mt_v[...],
            dimension_numbers=(((1,), (0,)), ((), ())),
            preferred_element_type=jnp.float32)

    for base in range(0, N_FULL, NBUF):
        for k in range(NBUF):
            i = base + k
            mslot = i % MBUF
            in_copy(i, mslot).wait()
            if i >= NBUF:
                out_copy(i - NBUF, k).wait()
            obufs[k][...] = matmul(mslot)
            out_copy(i, k).start(priority=k % 2)
            if i + MBUF < N_FULL:
                in_copy(i + MBUF, mslot).start()

    tail_in = pltpu.make_async_copy(
        mem_ref.at[pl.ds(N_FULL * N_TILE, N_TAIL), :],
        m_v.at[0, pl.ds(0, N_TAIL), :], in_sem.at[0])
    tail_in.start()
    tail_in.wait()
    mt_v[:, : N_TAIL] = m_v[0, : N_TAIL, :].T
    o_tail[...] = lax.dot_general(
        x, mt_v[:, : N_TAIL],
        dimension_numbers=(((1,), (0,)), ((), ())),
        preferred_element_type=jnp.float32)
    tail_out = pltpu.make_async_copy(
        o_tail, out_ref.at[:, pl.ds(N_FULL * N_TILE, N_TAIL)], tail_sem)
    tail_out.start()

    for i in range(N_FULL - NBUF, N_FULL):
        out_copy(i, i % NBUF).wait()
    tail_out.wait()


def kernel(inputs, targets, mem):
    del targets
    b, f = inputs.shape
    n = mem.shape[0]
    return pl.pallas_call(
        _body,
        in_specs=[
            pl.BlockSpec(memory_space=pltpu.VMEM),
            pl.BlockSpec(memory_space=pltpu.MemorySpace.HBM),
        ],
        out_specs=pl.BlockSpec(memory_space=pltpu.MemorySpace.HBM),
        out_shape=jax.ShapeDtypeStruct((b, n), jnp.float32),
        scratch_shapes=[
            pltpu.VMEM((MBUF, N_TILE, f), jnp.float32),
            pltpu.VMEM((f, N_TILE), jnp.float32),
            pltpu.VMEM((b, N_TILE), jnp.float32),
            pltpu.VMEM((b, N_TILE), jnp.float32),
            pltpu.VMEM((b, N_TILE), jnp.float32),
            pltpu.VMEM((b, N_TILE), jnp.float32),
            pltpu.VMEM((b, N_TAIL), jnp.float32),
            pltpu.SemaphoreType.DMA((MBUF,)),
            pltpu.SemaphoreType.DMA,
            pltpu.SemaphoreType.DMA,
            pltpu.SemaphoreType.DMA,
            pltpu.SemaphoreType.DMA,
            pltpu.SemaphoreType.DMA,
        ],
    )(inputs, mem)


# speedup vs baseline: 1.0112x; 1.0112x over previous
import jax
import jax.numpy as jnp
from jax import lax
from jax.experimental import pallas as pl
from jax.experimental.pallas import tpu as pltpu

N_TILE = 2048
N_FULL = 48
N_TAIL = 1696
NBUF = 4
N_STEPS = N_FULL // NBUF


def _body(x_ref, mem_ref, dummy_ref, out_ref,
          m_v, mt_v, o_v, m_tail, o_tail, in_sem, out_sem, tail_sem):
    x = x_ref[...]

    def in_copy(i, slot):
        return pltpu.make_async_copy(
            mem_ref.at[pl.ds(i * N_TILE, N_TILE), :], m_v.at[slot],
            in_sem.at[slot])

    def out_copy(i, slot):
        return pltpu.make_async_copy(
            o_v.at[slot], out_ref.at[:, pl.ds(i * N_TILE, N_TILE)],
            out_sem.at[slot])

    for s in range(NBUF):
        in_copy(s, s).start()

    def step(j, carry):
        for k in range(NBUF):
            i = j * NBUF + k
            in_copy(i, k).wait()

            @pl.when(j > 0)
            def _():
                out_copy(i - NBUF, k).wait()

            mt_v[...] = m_v[k].T
            o_v[k] = lax.dot_general(
                x, mt_v[...],
                dimension_numbers=(((1,), (0,)), ((), ())),
                preferred_element_type=jnp.float32)
            out_copy(i, k).start()

            @pl.when(j < N_STEPS - 1)
            def _():
                in_copy(i + NBUF, k).start()

        return carry

    lax.fori_loop(0, N_STEPS, step, 0)

    tail_in = pltpu.make_async_copy(
        mem_ref.at[pl.ds(N_FULL * N_TILE, N_TAIL), :], m_tail, tail_sem)
    tail_in.start()
    tail_in.wait()
    mt_v[:, : N_TAIL] = m_tail[...].T
    o_tail[...] = lax.dot_general(
        x, mt_v[:, : N_TAIL],
        dimension_numbers=(((1,), (0,)), ((), ())),
        preferred_element_type=jnp.float32)
    tail_out = pltpu.make_async_copy(
        o_tail, out_ref.at[:, pl.ds(N_FULL * N_TILE, N_TAIL)], tail_sem)
    tail_out.start()

    for k in range(NBUF):
        out_copy(N_FULL - NBUF + k, k).wait()
    tail_out.wait()


def kernel(inputs, targets, mem):
    del targets
    b, f = inputs.shape
    n = mem.shape[0]
    _, out = pl.pallas_call(
        _body,
        in_specs=[
            pl.BlockSpec(memory_space=pltpu.VMEM),
            pl.BlockSpec(memory_space=pltpu.MemorySpace.HBM),
        ],
        out_specs=[
            pl.BlockSpec(memory_space=pltpu.MemorySpace.HBM),
            pl.BlockSpec(memory_space=pltpu.MemorySpace.HBM),
        ],
        out_shape=[
            jax.ShapeDtypeStruct((b, n), jnp.float32),
            jax.ShapeDtypeStruct((b, n), jnp.float32),
        ],
        scratch_shapes=[
            pltpu.VMEM((NBUF, N_TILE, f), jnp.float32),
            pltpu.VMEM((f, N_TILE), jnp.float32),
            pltpu.VMEM((NBUF, b, N_TILE), jnp.float32),
            pltpu.VMEM((N_TAIL, f), jnp.float32),
            pltpu.VMEM((b, N_TAIL), jnp.float32),
            pltpu.SemaphoreType.DMA((NBUF,)),
            pltpu.SemaphoreType.DMA((NBUF,)),
            pltpu.SemaphoreType.DMA,
        ],
    )(inputs, mem)
    return out
